# baseline (speedup 1.0000x reference)
import functools

import jax
import jax.numpy as jnp
from jax import lax
from jax.experimental import pallas as pl
from jax.experimental.pallas import tpu as pltpu
from jax.experimental.pallas import tpu_sc as plsc

_VOCAB = 32000
_HID = 2048
_B = 4
_S = 2048

_NC = 2
_NS = 16
_NW = _NC * _NS
_NTOK = _B * _S
_TOK_PER_W = _NTOK // _NW
_CHUNK = 16
_NCHUNK = _TOK_PER_W // _CHUNK

_MASK_TI = 256
_F32_MIN = float(jnp.finfo(jnp.float32).min)


def _gather_body(ids_hbm, table_hbm, out_hbm, idx_v, buf0, buf1, sem0, sem1):
    wid = lax.axis_index("s") * _NC + lax.axis_index("c")
    base = wid * _TOK_PER_W
    pltpu.sync_copy(ids_hbm.at[pl.ds(base, _TOK_PER_W)], idx_v)

    bufs = (buf0, buf1)
    sems = (sem0, sem1)

    def start(c):
        return pltpu.async_copy(
            table_hbm.at[idx_v.at[pl.ds(c * _CHUNK, _CHUNK)]],
            bufs[c % 2],
            sems[c % 2],
        )

    copies = [None, None]
    copies[0] = start(0)
    for c in range(_NCHUNK):
        if c + 1 < _NCHUNK:
            copies[(c + 1) % 2] = start(c + 1)
        copies[c % 2].wait()
        pltpu.sync_copy(
            bufs[c % 2],
            out_hbm.at[pl.ds(base + c * _CHUNK, _CHUNK)],
        )


@jax.jit
def _sc_gather(ids, table):
    run = pl.kernel(
        _gather_body,
        out_type=jax.ShapeDtypeStruct((_NTOK, _HID), jnp.float32),
        mesh=plsc.VectorSubcoreMesh(core_axis_name="c", subcore_axis_name="s"),
        scratch_types=[
            pltpu.VMEM((_TOK_PER_W,), jnp.int32),
            pltpu.VMEM((_CHUNK, _HID), jnp.float32),
            pltpu.VMEM((_CHUNK, _HID), jnp.float32),
            pltpu.SemaphoreType.DMA,
            pltpu.SemaphoreType.DMA,
        ],
        name="sc_embedding_gather",
    )
    return run(ids, table)


def _mask_body(amask_ref, out_ref):
    ti = pl.program_id(1)
    rows = lax.broadcasted_iota(jnp.int32, (_MASK_TI, _S), 0) + ti * _MASK_TI
    cols = lax.broadcasted_iota(jnp.int32, (_MASK_TI, _S), 1)
    causal = jnp.where(cols > rows, _F32_MIN, 0.0).astype(jnp.float32)
    am = amask_ref[0, 0, :]
    pad = jnp.where(am > 0.0, 0.0, _F32_MIN).astype(jnp.float32)
    out_ref[0, 0, :, :] = causal + pad[None, :]


@jax.jit
def _tc_mask(amask_f32):
    return pl.pallas_call(
        _mask_body,
        grid=(_B, _S // _MASK_TI),
        in_specs=[pl.BlockSpec((1, 1, _S), lambda b, i: (b, 0, 0))],
        out_specs=pl.BlockSpec((1, 1, _MASK_TI, _S), lambda b, i: (b, 0, i, 0)),
        out_shape=jax.ShapeDtypeStruct((_B, 1, _S, _S), jnp.float32),
    )(amask_f32.reshape(_B, 1, _S))


def kernel(input_ids, attention_mask, emb_table):
    ids = input_ids.reshape(-1).astype(jnp.int32)
    combined = _tc_mask(attention_mask.astype(jnp.float32))
    emb = _sc_gather(ids, emb_table).reshape(_B, _S, _HID)
    return emb, combined


# speedup vs baseline: 1.0220x; 1.0220x over previous
import jax
import jax.numpy as jnp
from jax import lax
from jax.experimental import pallas as pl
from jax.experimental.pallas import tpu as pltpu
from jax.experimental.pallas import tpu_sc as plsc

_VOCAB = 32000
_HID = 2048
_B = 4
_S = 2048

_NC = 2
_NS = 16
_NW = _NC * _NS
_NTOK = _B * _S
_TOK_PER_W = _NTOK // _NW
_CHUNK = 16
_NCHUNK = _TOK_PER_W // _CHUNK

_MASK_TI = 256
_F32_MIN = float(jnp.finfo(jnp.float32).min)


_NBUF = 3


_W_PER_B = _NW // _B


def _gather_body(ids_hbm, table_hbm, out_hbm, idx_v, bufs, gsems, ssems):
    wid = lax.axis_index("s") * _NC + lax.axis_index("c")
    base = wid * _TOK_PER_W
    row = wid // _W_PER_B
    col0 = (wid % _W_PER_B) * _TOK_PER_W
    pltpu.sync_copy(ids_hbm.at[row, pl.ds(col0, _TOK_PER_W)], idx_v)

    def start_gather(c):
        return pltpu.async_copy(
            table_hbm.at[idx_v.at[pl.ds(c * _CHUNK, _CHUNK)]],
            bufs[c % _NBUF],
            gsems[c % _NBUF],
        )

    def start_out(c):
        return pltpu.async_copy(
            bufs[c % _NBUF],
            out_hbm.at[pl.ds(base + c * _CHUNK, _CHUNK)],
            ssems[c % _NBUF],
        )

    in_cp = [start_gather(c) for c in range(_NBUF)]
    out_cp = [None] * _NBUF
    for c in range(_NCHUNK):
        b = c % _NBUF
        nxt = c + _NBUF - 1
        if c >= 1 and nxt < _NCHUNK:
            nb = nxt % _NBUF
            out_cp[nb].wait()
            in_cp[nb] = start_gather(nxt)
        in_cp[b].wait()
        out_cp[b] = start_out(c)
    for c in range(_NCHUNK - _NBUF, _NCHUNK):
        out_cp[c % _NBUF].wait()


@jax.jit
def _sc_gather(ids2d, table):
    run = pl.kernel(
        _gather_body,
        out_type=jax.ShapeDtypeStruct((_NTOK, _HID), jnp.float32),
        mesh=plsc.VectorSubcoreMesh(core_axis_name="c", subcore_axis_name="s"),
        scratch_types=[
            pltpu.VMEM((_TOK_PER_W,), jnp.int32),
            [pltpu.VMEM((_CHUNK, _HID), jnp.float32) for _ in range(_NBUF)],
            [pltpu.SemaphoreType.DMA for _ in range(_NBUF)],
            [pltpu.SemaphoreType.DMA for _ in range(_NBUF)],
        ],
        name="sc_embedding_gather",
    )
    return run(ids2d, table)


def _mask_body(amask_ref, out_ref):
    ti = pl.program_id(1)
    rows = lax.broadcasted_iota(jnp.int32, (_MASK_TI, _S), 0) + ti * _MASK_TI
    cols = lax.broadcasted_iota(jnp.int32, (_MASK_TI, _S), 1)
    causal = jnp.where(cols > rows, _F32_MIN, 0.0).astype(jnp.float32)
    am = amask_ref[0, 0, :]
    pad = jnp.where(am > 0.0, 0.0, _F32_MIN).astype(jnp.float32)
    out_ref[0, 0, :, :] = causal + pad[None, :]


@jax.jit
def _tc_mask(amask_f32):
    return pl.pallas_call(
        _mask_body,
        grid=(_B, _S // _MASK_TI),
        in_specs=[pl.BlockSpec((1, 1, _S), lambda b, i: (b, 0, 0))],
        out_specs=pl.BlockSpec((1, 1, _MASK_TI, _S), lambda b, i: (b, 0, i, 0)),
        out_shape=jax.ShapeDtypeStruct((_B, 1, _S, _S), jnp.float32),
    )(amask_f32.reshape(_B, 1, _S))


def kernel(input_ids, attention_mask, emb_table):
    combined = _tc_mask(attention_mask.astype(jnp.float32))
    emb = _sc_gather(input_ids.astype(jnp.int32), emb_table)
    return emb.reshape(_B, _S, _HID), combined
